# baseline (speedup 1.0000x reference)
import functools

import jax
import jax.numpy as jnp
from jax import lax
from jax.experimental import pallas as pl
from jax.experimental.pallas import tpu as pltpu
from jax.experimental.pallas import tpu_sc as plsc

N = 10000
DEG = 32
D_IN = 128
D_OUT = 128
D_BOND = 16
H = 8
HD = D_OUT // H
E = N * DEG

_NC = 2
_NS = 16
_NW = _NC * _NS
_PW = E // _NW
_CH = 80
_NCHUNK = _PW // _CH


def _pre_body(x_ref, w_ref, b_ref, o_ref):
    o_ref[...] = lax.dot_general(
        x_ref[...], w_ref[...],
        (((1,), (1,)), ((), ())),
        preferred_element_type=jnp.float32,
        precision=lax.Precision.HIGHEST,
    ) + b_ref[...]


def _pre_call(x, w, b):
    blk = 1000
    return pl.pallas_call(
        _pre_body,
        grid=(N // blk,),
        in_specs=[
            pl.BlockSpec((blk, D_IN), lambda i: (i, 0)),
            pl.BlockSpec((D_OUT, D_IN), lambda i: (0, 0)),
            pl.BlockSpec((1, D_OUT), lambda i: (0, 0)),
        ],
        out_specs=pl.BlockSpec((blk, D_OUT), lambda i: (i, 0)),
        out_shape=jax.ShapeDtypeStruct((N, D_OUT), jnp.float32),
    )(x, w, b)


_NBUF = 4


def _make_ring_body(width):
    def body(idx_hbm, table_hbm, out_hbm, *scratch):
        ia = list(scratch[0:_NBUF])
        ra = list(scratch[_NBUF:2 * _NBUF])
        sga = list(scratch[2 * _NBUF:3 * _NBUF])
        soa = list(scratch[3 * _NBUF:4 * _NBUF])
        wid = lax.axis_index("s") * _NC + lax.axis_index("c")
        base = wid * _PW

        def load_idx(c, b):
            off = pl.multiple_of(base + c * _CH, 8)
            pltpu.sync_copy(idx_hbm.at[pl.ds(off, _CH)], ia[b])

        def start_gather(b):
            pltpu.async_copy(table_hbm.at[ia[b]], ra[b], sga[b])

        for b in range(_NBUF):
            load_idx(b, b)
            start_gather(b)

        def loop_body(i, carry):
            for b in range(_NBUF):
                c = i * _NBUF + b
                off = pl.multiple_of(base + c * _CH, 8)

                @pl.when(c < _NCHUNK)
                def _():
                    pltpu.make_async_copy(table_hbm.at[ia[b]], ra[b],
                                          sga[b]).wait()
                    pltpu.async_copy(ra[b], out_hbm.at[pl.ds(off, _CH)],
                                     soa[b])

                    @pl.when(c + _NBUF < _NCHUNK)
                    def _():
                        load_idx(c + _NBUF, b)
                        pltpu.make_async_copy(
                            ra[b], out_hbm.at[pl.ds(off, _CH)], soa[b]).wait()
                        start_gather(b)

                    @pl.when(c + _NBUF >= _NCHUNK)
                    def _():
                        pltpu.make_async_copy(
                            ra[b], out_hbm.at[pl.ds(off, _CH)], soa[b]).wait()
            return carry

        lax.fori_loop(0, (_NCHUNK + _NBUF - 1) // _NBUF, loop_body, 0)

    return body


def _sc_gather_rows(idx_flat, table, width, tc_tiling):
    mesh = plsc.VectorSubcoreMesh(core_axis_name="c", subcore_axis_name="s")
    scratch = (
        [pltpu.VMEM((_CH,), jnp.int32) for _ in range(_NBUF)]
        + [pltpu.VMEM((_CH, width), jnp.float32) for _ in range(_NBUF)]
        + [pltpu.SemaphoreType.DMA for _ in range(2 * _NBUF)]
    )
    fn = functools.partial(
        pl.kernel,
        mesh=mesh,
        compiler_params=pltpu.CompilerParams(use_tc_tiling_on_sc=tc_tiling),
        out_type=jax.ShapeDtypeStruct((E, width), jnp.float32),
        scratch_types=scratch,
    )(_make_ring_body(width))
    return fn(idx_flat, table)


def _sc_gather(a2a_flat, a2b_flat, h_atoms, edge_attr):
    ga = _sc_gather_rows(a2a_flat, h_atoms, D_OUT, True)
    ge = _sc_gather_rows(a2b_flat, edge_attr, D_BOND, False)
    return ga, ge


def _dot(a, b, prec=lax.Precision.DEFAULT):
    return lax.dot_general(a, b, (((1,), (0,)), ((), ())),
                           preferred_element_type=jnp.float32,
                           precision=prec)


def _post_body(ha_ref, ga_ref, ge_ref, a2a_ref,
               gmat_ref, cw_ref, dmat_ref, smat_ref, rmat_ref,
               wbw_ref, wbb_ref, out_ref):
    blk = ha_ref.shape[0]
    ga = ga_ref[...]
    ge = ge_ref[...]
    gb = _dot(ge, wbw_ref[...], lax.Precision.DEFAULT) + wbb_ref[...]
    val = ga + gb
    sz = _dot(ga, dmat_ref[...])
    sw = _dot(ge, gmat_ref[...]) + cw_ref[...]
    src = _dot(ha_ref[...], smat_ref[...])
    scores = jnp.reshape(sz + sw, (blk, DEG, H)) + src[:, None, :]
    scores = jnp.where(scores >= 0, scores, 0.2 * scores)
    a2a3 = jnp.reshape(a2a_ref[...], (blk, DEG, 1))
    scores = jnp.where(a2a3 != 0, scores, jnp.float32(-1e9))
    m = jnp.max(scores, axis=1, keepdims=True)
    ex = jnp.exp(scores - m)
    alpha = ex / jnp.sum(ex, axis=1, keepdims=True)
    arep = _dot(jnp.reshape(alpha, (blk * DEG, H)), rmat_ref[...],
                lax.Precision.DEFAULT)
    out_ref[...] = jnp.sum(jnp.reshape(arep * val, (blk, DEG, D_OUT)), axis=1)


def _post_call(h_atoms, ga, ge, a2a, gmat, cw, dmat, smat, rmat, wbw, wbb):
    blk = 400
    return pl.pallas_call(
        _post_body,
        grid=(N // blk,),
        in_specs=[
            pl.BlockSpec((blk, D_OUT), lambda i: (i, 0)),
            pl.BlockSpec((blk * DEG, D_OUT), lambda i: (i, 0)),
            pl.BlockSpec((blk * DEG, D_BOND), lambda i: (i, 0)),
            pl.BlockSpec((blk, DEG), lambda i: (i, 0)),
            pl.BlockSpec((D_BOND, H), lambda i: (0, 0)),
            pl.BlockSpec((1, H), lambda i: (0, 0)),
            pl.BlockSpec((D_OUT, H), lambda i: (0, 0)),
            pl.BlockSpec((D_OUT, H), lambda i: (0, 0)),
            pl.BlockSpec((H, D_OUT), lambda i: (0, 0)),
            pl.BlockSpec((D_BOND, D_OUT), lambda i: (0, 0)),
            pl.BlockSpec((1, D_OUT), lambda i: (0, 0)),
        ],
        out_specs=pl.BlockSpec((blk, D_OUT), lambda i: (i, 0)),
        out_shape=jax.ShapeDtypeStruct((N, D_OUT), jnp.float32),
    )(h_atoms, ga, ge, a2a, gmat, cw, dmat, smat, rmat, wbw, wbb)


def kernel(x, edge_attr, a2a, a2b, W_atom_w, W_atom_b, W_bond_w, W_bond_b,
           att_src, att_dst, att_edge):
    a2a = a2a.astype(jnp.int32)
    a2b = a2b.astype(jnp.int32)
    h_atoms = _pre_call(x, W_atom_w, W_atom_b.reshape(1, D_OUT))
    ga, ge = _sc_gather(a2a.reshape(E), a2b.reshape(E), h_atoms, edge_attr)

    eye = jnp.eye(H, dtype=jnp.float32)
    def _blockdiag(att):
        a = att.reshape(H, HD)
        return (eye[:, :, None] * a[:, None, :]).transpose(0, 2, 1).reshape(D_OUT, H)
    dmat = _blockdiag(att_dst)
    amat = _blockdiag(att_edge)
    smat = _blockdiag(att_src)
    rmat = jnp.repeat(eye, HD, axis=1)
    gmat = W_bond_w.T @ amat
    cw = W_bond_b.reshape(1, D_OUT) @ amat

    out = _post_call(h_atoms, ga, ge, a2a, gmat, cw, dmat, smat, rmat,
                     W_bond_w.T, W_bond_b.reshape(1, D_OUT))
    return out


# speedup vs baseline: 1.1958x; 1.1958x over previous
import functools

import jax
import jax.numpy as jnp
from jax import lax
from jax.experimental import pallas as pl
from jax.experimental.pallas import tpu as pltpu
from jax.experimental.pallas import tpu_sc as plsc

N = 10000
DEG = 32
D_IN = 128
D_OUT = 128
D_BOND = 16
H = 8
HD = D_OUT // H
E = N * DEG

_NC = 2
_NS = 16
_NW = _NC * _NS
_PW = E // _NW
_CH = 80
_NCHUNK = _PW // _CH


def _pre_body(x_ref, w_ref, b_ref, o_ref):
    o_ref[...] = lax.dot_general(
        x_ref[...], w_ref[...],
        (((1,), (1,)), ((), ())),
        preferred_element_type=jnp.float32,
        precision=lax.Precision.HIGHEST,
    ) + b_ref[...]


def _pre_call(x, w, b):
    blk = 1000
    return pl.pallas_call(
        _pre_body,
        grid=(N // blk,),
        in_specs=[
            pl.BlockSpec((blk, D_IN), lambda i: (i, 0)),
            pl.BlockSpec((D_OUT, D_IN), lambda i: (0, 0)),
            pl.BlockSpec((1, D_OUT), lambda i: (0, 0)),
        ],
        out_specs=pl.BlockSpec((blk, D_OUT), lambda i: (i, 0)),
        out_shape=jax.ShapeDtypeStruct((N, D_OUT), jnp.float32),
    )(x, w, b)


_NBUF = 4


def _make_ring_body(width):
    def body(idx_hbm, table_hbm, out_hbm, *scratch):
        ia = list(scratch[0:_NBUF])
        ra = list(scratch[_NBUF:2 * _NBUF])
        sga = list(scratch[2 * _NBUF:3 * _NBUF])
        soa = list(scratch[3 * _NBUF:4 * _NBUF])
        wid = lax.axis_index("s") * _NC + lax.axis_index("c")
        base = wid * _PW

        def load_idx(c, b):
            off = pl.multiple_of(base + c * _CH, 8)
            pltpu.sync_copy(idx_hbm.at[pl.ds(off, _CH)], ia[b])

        def start_gather(b):
            pltpu.async_copy(table_hbm.at[ia[b]], ra[b], sga[b])

        for b in range(_NBUF):
            load_idx(b, b)
            start_gather(b)

        def loop_body(i, carry):
            for b in range(_NBUF):
                c = i * _NBUF + b
                off = pl.multiple_of(base + c * _CH, 8)

                @pl.when(c < _NCHUNK)
                def _():
                    pltpu.make_async_copy(table_hbm.at[ia[b]], ra[b],
                                          sga[b]).wait()
                    pltpu.async_copy(ra[b], out_hbm.at[pl.ds(off, _CH)],
                                     soa[b])

                    @pl.when(c + _NBUF < _NCHUNK)
                    def _():
                        load_idx(c + _NBUF, b)
                        pltpu.make_async_copy(
                            ra[b], out_hbm.at[pl.ds(off, _CH)], soa[b]).wait()
                        start_gather(b)

                    @pl.when(c + _NBUF >= _NCHUNK)
                    def _():
                        pltpu.make_async_copy(
                            ra[b], out_hbm.at[pl.ds(off, _CH)], soa[b]).wait()
            return carry

        lax.fori_loop(0, (_NCHUNK + _NBUF - 1) // _NBUF, loop_body, 0)

    return body


def _sc_gather_rows(idx_flat, table, width, tc_tiling):
    mesh = plsc.VectorSubcoreMesh(core_axis_name="c", subcore_axis_name="s")
    scratch = (
        [pltpu.VMEM((_CH,), jnp.int32) for _ in range(_NBUF)]
        + [pltpu.VMEM((_CH, width), jnp.float32) for _ in range(_NBUF)]
        + [pltpu.SemaphoreType.DMA for _ in range(2 * _NBUF)]
    )
    fn = functools.partial(
        pl.kernel,
        mesh=mesh,
        compiler_params=pltpu.CompilerParams(use_tc_tiling_on_sc=tc_tiling),
        out_type=jax.ShapeDtypeStruct((E, width), jnp.float32),
        scratch_types=scratch,
    )(_make_ring_body(width))
    return fn(idx_flat, table)


def _sc_gather(a2a_flat, a2b_flat, h_atoms, edge_attr):
    ga = _sc_gather_rows(a2a_flat, h_atoms, D_OUT, True)
    ge = _sc_gather_rows(a2b_flat, edge_attr, D_BOND, False)
    return ga, ge


def _dot(a, b, prec=lax.Precision.DEFAULT):
    return lax.dot_general(a, b, (((1,), (0,)), ((), ())),
                           preferred_element_type=jnp.float32,
                           precision=prec)


def _post_body(ha_ref, ga_ref, ge_ref, a2a_ref,
               wbig_ref, amat_ref, dmat_ref, smat_ref, rmat_ref,
               wbb_ref, out_ref):
    blk = ha_ref.shape[0]
    ga = ga_ref[...]
    ge128 = jnp.reshape(ge_ref[...], (blk * DEG // 8, 128))
    gbw = _dot(ge128, wbig_ref[...])
    gb = jnp.reshape(gbw, (blk * DEG, D_OUT)) + wbb_ref[...]
    val = ga + gb
    sz = _dot(ga, dmat_ref[...])
    sw = _dot(gb, amat_ref[...])
    src = _dot(ha_ref[...], smat_ref[...])
    scores = jnp.reshape(sz + sw, (blk, DEG, H)) + src[:, None, :]
    scores = jnp.where(scores >= 0, scores, 0.2 * scores)
    a2a3 = jnp.reshape(a2a_ref[...], (blk, DEG, 1))
    scores = jnp.where(a2a3 != 0, scores, jnp.float32(-1e9))
    m = jnp.max(scores, axis=1, keepdims=True)
    ex = jnp.exp(scores - m)
    alpha = ex / jnp.sum(ex, axis=1, keepdims=True)
    arep = _dot(jnp.reshape(alpha, (blk * DEG, H)), rmat_ref[...],
                lax.Precision.DEFAULT)
    out_ref[...] = jnp.sum(jnp.reshape(arep * val, (blk, DEG, D_OUT)), axis=1)


def _post_call(h_atoms, ga, ge_flat, a2a, wbig, amat, dmat, smat, rmat, wbb):
    blk = 200
    return pl.pallas_call(
        _post_body,
        grid=(N // blk,),
        in_specs=[
            pl.BlockSpec((blk, D_OUT), lambda i: (i, 0)),
            pl.BlockSpec((blk * DEG, D_OUT), lambda i: (i, 0)),
            pl.BlockSpec((blk * DEG * D_BOND,), lambda i: (i,)),
            pl.BlockSpec((blk, DEG), lambda i: (i, 0)),
            pl.BlockSpec((D_OUT, 1024), lambda i: (0, 0)),
            pl.BlockSpec((D_OUT, H), lambda i: (0, 0)),
            pl.BlockSpec((D_OUT, H), lambda i: (0, 0)),
            pl.BlockSpec((D_OUT, H), lambda i: (0, 0)),
            pl.BlockSpec((H, D_OUT), lambda i: (0, 0)),
            pl.BlockSpec((1, D_OUT), lambda i: (0, 0)),
        ],
        out_specs=pl.BlockSpec((blk, D_OUT), lambda i: (i, 0)),
        out_shape=jax.ShapeDtypeStruct((N, D_OUT), jnp.float32),
    )(h_atoms, ga, ge_flat, a2a, wbig, amat, dmat, smat, rmat, wbb)


def kernel(x, edge_attr, a2a, a2b, W_atom_w, W_atom_b, W_bond_w, W_bond_b,
           att_src, att_dst, att_edge):
    a2a = a2a.astype(jnp.int32)
    a2b = a2b.astype(jnp.int32)
    h_atoms = _pre_call(x, W_atom_w, W_atom_b.reshape(1, D_OUT))
    ga, ge = _sc_gather(a2a.reshape(E), a2b.reshape(E), h_atoms, edge_attr)

    eye = jnp.eye(H, dtype=jnp.float32)
    def _blockdiag(att):
        a = att.reshape(H, HD)
        return (eye[:, :, None] * a[:, None, :]).transpose(0, 2, 1).reshape(D_OUT, H)
    dmat = _blockdiag(att_dst)
    amat = _blockdiag(att_edge)
    smat = _blockdiag(att_src)
    rmat = jnp.repeat(eye, HD, axis=1)
    wbig = jnp.einsum('pq,jf->pjqf', eye, W_bond_w.T).reshape(D_OUT, 8 * D_OUT)

    out = _post_call(h_atoms, ga, ge.reshape(E * D_BOND), a2a, wbig, amat,
                     dmat, smat, rmat, W_bond_b.reshape(1, D_OUT))
    return out
